# Initial kernel scaffold; baseline (speedup 1.0000x reference)
import jax
import jax.numpy as jnp
from jax.experimental import pallas as pl


def kernel(x):
    raise NotImplementedError("write your pallas kernel here")


# speedup vs baseline: 2.7359x; 2.7359x over previous
import jax
import jax.numpy as jnp
from jax.experimental import pallas as pl
from jax.experimental.pallas import tpu as pltpu

_NUM = 10
_HBLK = 256


def _thermo_block(x_ref, o_ref):
    x = x_ref[0]
    k = jax.lax.broadcasted_iota(jnp.int32, (_NUM, 1, 1), 0)
    thr = jnp.where(k == 0, -1.0, k.astype(jnp.float32) / _NUM)
    o_ref[0] = (x[None, :, :] > thr).astype(x.dtype)


def kernel(x):
    B, C, H, W = x.shape
    xf = x.reshape(B * C, H, W)
    out = pl.pallas_call(
        _thermo_block,
        grid=(B * C, H // _HBLK),
        in_specs=[pl.BlockSpec((1, _HBLK, W), lambda i, j: (i, j, 0))],
        out_specs=pl.BlockSpec((1, _NUM, _HBLK, W), lambda i, j: (i, 0, j, 0)),
        out_shape=jax.ShapeDtypeStruct((B * C, _NUM, H, W), x.dtype),
        compiler_params=pltpu.CompilerParams(
            dimension_semantics=("parallel", "parallel"),
        ),
    )(xf)
    return out.reshape(B, C * _NUM, H, W)


# speedup vs baseline: 2.8427x; 1.0391x over previous
import jax
import jax.numpy as jnp
from jax.experimental import pallas as pl
from jax.experimental.pallas import tpu as pltpu

_NUM = 10
_HBLK = 512


def _thermo_block(x_ref, o_ref):
    x = x_ref[0]
    k = jax.lax.broadcasted_iota(jnp.int32, (_NUM, 1, 1), 0)
    thr = jnp.where(k == 0, -1.0, k.astype(jnp.float32) / _NUM)
    o_ref[0] = (x[None, :, :] > thr).astype(x.dtype)


def kernel(x):
    B, C, H, W = x.shape
    xf = x.reshape(B * C, H, W)
    out = pl.pallas_call(
        _thermo_block,
        grid=(B * C, H // _HBLK),
        in_specs=[pl.BlockSpec((1, _HBLK, W), lambda i, j: (i, j, 0))],
        out_specs=pl.BlockSpec((1, _NUM, _HBLK, W), lambda i, j: (i, 0, j, 0)),
        out_shape=jax.ShapeDtypeStruct((B * C, _NUM, H, W), x.dtype),
        compiler_params=pltpu.CompilerParams(
            dimension_semantics=("parallel", "parallel"),
        ),
    )(xf)
    return out.reshape(B, C * _NUM, H, W)


# speedup vs baseline: 2.8843x; 1.0146x over previous
import jax
import jax.numpy as jnp
from jax.experimental import pallas as pl
from jax.experimental.pallas import tpu as pltpu

_NUM = 10
_CBLK = 2


def _thermo_block(x_ref, o_ref):
    x = x_ref[...]
    k = jax.lax.broadcasted_iota(jnp.int32, (1, _NUM, 1, 1), 1)
    thr = jnp.where(k == 0, -1.0, k.astype(jnp.float32) / _NUM)
    o_ref[...] = (x[:, None, :, :] > thr).astype(x.dtype)


def kernel(x):
    B, C, H, W = x.shape
    xf = x.reshape(B * C, H, W)
    out = pl.pallas_call(
        _thermo_block,
        grid=(B * C // _CBLK,),
        in_specs=[pl.BlockSpec((_CBLK, H, W), lambda i: (i, 0, 0))],
        out_specs=pl.BlockSpec((_CBLK, _NUM, H, W), lambda i: (i, 0, 0, 0)),
        out_shape=jax.ShapeDtypeStruct((B * C, _NUM, H, W), x.dtype),
        compiler_params=pltpu.CompilerParams(
            dimension_semantics=("parallel",),
            vmem_limit_bytes=56 * 1024 * 1024,
        ),
    )(xf)
    return out.reshape(B, C * _NUM, H, W)
